# baseline (speedup 1.0000x reference)
import jax
import jax.numpy as jnp
from jax import lax
from jax.experimental import pallas as pl
from jax.experimental.pallas import tpu as pltpu

NQ = 4
K = 1024
D = 256
BN = 512


def _rvq_block(x_ref, cb_ref, q_ref, idx_ref, loss_ref):
    r0 = x_ref[...]
    r = r0
    qsum = jnp.zeros((BN, D), jnp.float32)
    iota_k = lax.broadcasted_iota(jnp.int32, (BN, K), 1)
    idx_cols = lax.broadcasted_iota(jnp.int32, (BN, 8), 1)
    ones_row = jnp.ones((8, D), jnp.float32)
    idx_acc = jnp.zeros((BN, 8), jnp.int32)
    loss_rows = lax.broadcasted_iota(jnp.int32, (8, 128), 0)
    loss_cols = lax.broadcasted_iota(jnp.int32, (8, 128), 1)
    loss_contrib = jnp.zeros((8, 128), jnp.float32)

    for s in range(NQ):
        cb = cb_ref[s]
        rn = jnp.sum(r * r, axis=1, keepdims=True)
        cn = lax.dot_general(ones_row, cb * cb,
                             (((1,), (1,)), ((), ())),
                             preferred_element_type=jnp.float32,
                             precision=lax.Precision.HIGHEST)
        r2b = (2.0 * r).astype(jnp.bfloat16)
        cb_hi = cb.astype(jnp.bfloat16)
        m2 = lax.dot_general(r2b, cb_hi, (((1,), (1,)), ((), ())),
                             preferred_element_type=jnp.float32)
        d2 = (rn - m2) + cn[0:1, :]
        dmin = jnp.min(d2, axis=1, keepdims=True)
        idx = jnp.min(jnp.where(d2 == dmin, iota_k, K), axis=1,
                      keepdims=True)
        onehot = (iota_k == idx).astype(jnp.bfloat16)
        cb_hi_f = cb_hi.astype(jnp.float32)
        rem = cb - cb_hi_f
        cb_mid = rem.astype(jnp.bfloat16)
        cb_lo = (rem - cb_mid.astype(jnp.float32)).astype(jnp.bfloat16)
        dn = (((1,), (0,)), ((), ()))
        q = ((lax.dot_general(onehot, cb_hi, dn,
                              preferred_element_type=jnp.float32)
              + lax.dot_general(onehot, cb_mid, dn,
                                preferred_element_type=jnp.float32))
             + lax.dot_general(onehot, cb_lo, dn,
                               preferred_element_type=jnp.float32))
        r = r - q
        qsum = qsum + q
        loss_s = jnp.sum(r * r)
        loss_contrib = loss_contrib + loss_s * jnp.where(
            (loss_rows == 0) & (loss_cols == s), 1.0, 0.0)
        idx_acc = idx_acc + jnp.where(idx_cols == s, idx, 0)

    q_ref[...] = r0 + (qsum - r0)
    idx_ref[...] = idx_acc
    loss_ref[...] = loss_contrib[None]


def kernel(x, codebooks):
    B, T, Dd = x.shape
    N = B * T
    G = N // BN
    flat = x.reshape(N, Dd)
    q_flat, idx_pack, loss_sums = pl.pallas_call(
        _rvq_block,
        grid=(G,),
        in_specs=[
            pl.BlockSpec((BN, D), lambda i: (i, 0)),
            pl.BlockSpec((NQ, K, D), lambda i: (0, 0, 0)),
        ],
        out_specs=[
            pl.BlockSpec((BN, D), lambda i: (i, 0)),
            pl.BlockSpec((BN, 8), lambda i: (i, 0)),
            pl.BlockSpec((1, 8, 128), lambda i: (i, 0, 0)),
        ],
        out_shape=[
            jax.ShapeDtypeStruct((N, D), jnp.float32),
            jax.ShapeDtypeStruct((N, 8), jnp.int32),
            jax.ShapeDtypeStruct((G, 8, 128), jnp.float32),
        ],
        compiler_params=pltpu.CompilerParams(
            dimension_semantics=("parallel",),
        ),
    )(flat, codebooks)
    quantized = q_flat.reshape(B, T, Dd)
    indices = idx_pack[:, :NQ].reshape(B, T, NQ)
    vq_loss = 1.25 * jnp.sum(loss_sums[:, 0, :NQ]) / jnp.float32(N * Dd)
    losses = jnp.full((NQ,), vq_loss, dtype=jnp.float32)
    return quantized, indices, losses


# speedup vs baseline: 1.1855x; 1.1855x over previous
import jax
import jax.numpy as jnp
from jax import lax
from jax.experimental import pallas as pl
from jax.experimental.pallas import tpu as pltpu

NQ = 4
K = 1024
D = 256
BN = 1024


def _prep_block(cb_ref, hi_ref, cat_ref, cn_ref):
    ones_row = jnp.ones((8, D), jnp.float32)
    for s in range(NQ):
        cb = cb_ref[s]
        hi = cb.astype(jnp.bfloat16)
        rem = cb - hi.astype(jnp.float32)
        mid = rem.astype(jnp.bfloat16)
        lo = (rem - mid.astype(jnp.float32)).astype(jnp.bfloat16)
        hi_ref[s] = hi
        cat_ref[s, :, 0:D] = hi
        cat_ref[s, :, D:2 * D] = mid
        cat_ref[s, :, 2 * D:] = lo
        cn_ref[s] = lax.dot_general(ones_row, cb * cb,
                                    (((1,), (1,)), ((), ())),
                                    preferred_element_type=jnp.float32,
                                    precision=lax.Precision.HIGHEST)


def _rvq_block(x_ref, hi_ref, cat_ref, cn_ref,
               q_ref, idx_ref, loss_ref):
    r0 = x_ref[...]
    r = r0
    rn = jnp.sum(r * r, axis=1, keepdims=True)
    qsum = jnp.zeros((BN, D), jnp.float32)
    iota_f = lax.broadcasted_iota(jnp.int32, (BN, K), 1).astype(jnp.float32)
    idx_cols = lax.broadcasted_iota(jnp.int32, (BN, 8), 1)
    idx_acc = jnp.zeros((BN, 8), jnp.int32)
    loss_rows = lax.broadcasted_iota(jnp.int32, (8, 128), 0)
    loss_cols = lax.broadcasted_iota(jnp.int32, (8, 128), 1)
    loss_contrib = jnp.zeros((8, 128), jnp.float32)

    for s in range(NQ):
        cb_hi = hi_ref[s]
        r2b = (2.0 * r).astype(jnp.bfloat16)
        m2 = lax.dot_general(r2b, cb_hi, (((1,), (1,)), ((), ())),
                             preferred_element_type=jnp.float32)
        d2 = (rn - m2) + cn_ref[s, 0:1, :]
        dmin = jnp.min(d2, axis=1, keepdims=True)
        idx_f = jnp.min(jnp.where(d2 == dmin, iota_f, jnp.float32(K)),
                        axis=1, keepdims=True)
        onehot = (iota_f == idx_f).astype(jnp.bfloat16)
        dn = (((1,), (0,)), ((), ()))
        u = lax.dot_general(onehot, cat_ref[s], dn,
                            preferred_element_type=jnp.float32)
        q = (u[:, 0:D] + u[:, D:2 * D]) + u[:, 2 * D:]
        r = r - q
        qsum = qsum + q
        rn = jnp.sum(r * r, axis=1, keepdims=True)
        loss_s = jnp.sum(rn)
        loss_contrib = loss_contrib + loss_s * jnp.where(
            (loss_rows == 0) & (loss_cols == s), 1.0, 0.0)
        idx_acc = idx_acc + jnp.where(idx_cols == s,
                                      idx_f.astype(jnp.int32), 0)

    q_ref[...] = r0 + (qsum - r0)
    idx_ref[...] = idx_acc
    loss_ref[...] = loss_contrib[None]


def kernel(x, codebooks):
    B, T, Dd = x.shape
    N = B * T
    G = N // BN
    flat = x.reshape(N, Dd)

    cb_hi, cb_cat, cn = pl.pallas_call(
        _prep_block,
        in_specs=[pl.BlockSpec((NQ, K, D), lambda: (0, 0, 0))],
        out_specs=[
            pl.BlockSpec((NQ, K, D), lambda: (0, 0, 0)),
            pl.BlockSpec((NQ, K, 3 * D), lambda: (0, 0, 0)),
            pl.BlockSpec((NQ, 8, K), lambda: (0, 0, 0)),
        ],
        out_shape=[
            jax.ShapeDtypeStruct((NQ, K, D), jnp.bfloat16),
            jax.ShapeDtypeStruct((NQ, K, 3 * D), jnp.bfloat16),
            jax.ShapeDtypeStruct((NQ, 8, K), jnp.float32),
        ],
    )(codebooks)

    q_flat, idx_pack, loss_sums = pl.pallas_call(
        _rvq_block,
        grid=(G,),
        in_specs=[
            pl.BlockSpec((BN, D), lambda i: (i, 0)),
            pl.BlockSpec((NQ, K, D), lambda i: (0, 0, 0)),
            pl.BlockSpec((NQ, K, 3 * D), lambda i: (0, 0, 0)),
            pl.BlockSpec((NQ, 8, K), lambda i: (0, 0, 0)),
        ],
        out_specs=[
            pl.BlockSpec((BN, D), lambda i: (i, 0)),
            pl.BlockSpec((BN, 8), lambda i: (i, 0)),
            pl.BlockSpec((1, 8, 128), lambda i: (i, 0, 0)),
        ],
        out_shape=[
            jax.ShapeDtypeStruct((N, D), jnp.float32),
            jax.ShapeDtypeStruct((N, 8), jnp.int32),
            jax.ShapeDtypeStruct((G, 8, 128), jnp.float32),
        ],
        compiler_params=pltpu.CompilerParams(
            dimension_semantics=("parallel",),
        ),
    )(flat, cb_hi, cb_cat, cn)

    quantized = q_flat.reshape(B, T, Dd)
    indices = idx_pack[:, :NQ].reshape(B, T, NQ)
    vq_loss = 1.25 * jnp.sum(loss_sums[:, 0, :NQ]) / jnp.float32(N * Dd)
    losses = jnp.full((NQ,), vq_loss, dtype=jnp.float32)
    return quantized, indices, losses
